# baseline (speedup 1.0000x reference)
import functools

import jax
import jax.numpy as jnp
from jax import lax
from jax.experimental import pallas as pl
from jax.experimental.pallas import tpu as pltpu
from jax.experimental.pallas import tpu_sc as plsc

EMB_DIM = 64
NEG = 20
T = NEG + 1
LANES = 16
NC, NS = 2, 16
NW = NC * NS
C = 16
GB = 128
VBLK = 8192


def _pack_index(r):
    return (r >> 13) * (VBLK // 2) + (r & (VBLK // 2 - 1))


def _compact(wt):
    rows = wt.shape[1]
    grid = pl.cdiv(rows, VBLK)

    def body(in_ref, out_ref):
        x = in_ref[...]
        out_ref[:, 0:EMB_DIM] = x[:, 0:VBLK // 2].T
        out_ref[:, EMB_DIM:2 * EMB_DIM] = x[:, VBLK // 2:VBLK].T

    return pl.pallas_call(
        body,
        grid=(grid,),
        in_specs=[pl.BlockSpec((EMB_DIM, VBLK), lambda g: (0, g))],
        out_specs=pl.BlockSpec((VBLK // 2, 2 * EMB_DIM), lambda g: (g, 0)),
        out_shape=jax.ShapeDtypeStruct((grid * (VBLK // 2), 2 * EMB_DIM),
                                       jnp.float32),
    )(wt)


def _sc_scores(pu_raw, pu_pack, t_raw, t_pack, uc, vc, B):
    PW = B // NW
    NCHUNK = PW // C
    CT = C * T

    mesh = plsc.VectorSubcoreMesh(core_axis_name="c", subcore_axis_name="s")

    @functools.partial(
        pl.kernel,
        out_type=jax.ShapeDtypeStruct((B * T,), jnp.float32),
        mesh=mesh,
        scratch_types=[
            pltpu.VMEM((PW,), jnp.int32),
            pltpu.VMEM((PW,), jnp.int32),
            pltpu.VMEM((PW * T,), jnp.int32),
            pltpu.VMEM((PW * T,), jnp.int32),
            pltpu.VMEM((2, C, 2 * EMB_DIM), jnp.float32),
            pltpu.VMEM((2, CT, 2 * EMB_DIM), jnp.float32),
            pltpu.VMEM((CT,), jnp.float32),
            pltpu.SemaphoreType.DMA,
            pltpu.SemaphoreType.DMA,
        ],
        compiler_params=pltpu.CompilerParams(
            needs_layout_passes=False, use_tc_tiling_on_sc=True),
    )
    def k(puraw_hbm, pupack_hbm, traw_hbm, tpack_hbm, uc_hbm, vc_hbm, out_hbm,
          uraw_v, upack_v, traw_v, tpack_v, ubuf_v, tbuf_v, sc_v, sem_u, sem_t):
        wid = lax.axis_index("s") * NC + lax.axis_index("c")
        base = wid * PW
        pltpu.sync_copy(puraw_hbm.at[pl.ds(base, PW)], uraw_v)
        pltpu.sync_copy(pupack_hbm.at[pl.ds(base, PW)], upack_v)
        pltpu.sync_copy(traw_hbm.at[pl.ds(base * T, PW * T)], traw_v)
        pltpu.sync_copy(tpack_hbm.at[pl.ds(base * T, PW * T)], tpack_v)

        lane = lax.iota(jnp.int32, 16)
        m5 = lane < (T - LANES)
        nfull, rem = CT // GB, CT % GB

        def t_copies(cix, bank):
            cs = []
            for b in range(nfull):
                cs.append(pltpu.make_async_copy(
                    vc_hbm.at[tpack_v.at[pl.ds(cix * CT + b * GB, GB)]],
                    tbuf_v.at[bank, pl.ds(b * GB, GB)], sem_t))
            if rem:
                cs.append(pltpu.make_async_copy(
                    vc_hbm.at[tpack_v.at[pl.ds(cix * CT + nfull * GB, rem)]],
                    tbuf_v.at[bank, pl.ds(nfull * GB, rem)], sem_t))
            return cs

        def u_copy(cix, bank):
            return pltpu.make_async_copy(
                uc_hbm.at[upack_v.at[pl.ds(cix * C, C)]],
                ubuf_v.at[bank], sem_u)

        for cp in t_copies(0, 0):
            cp.start()
        u_copy(0, 0).start()

        def chunk_body(cix, carry):
            bank = lax.rem(cix, 2)
            nbank = 1 - bank

            @pl.when(cix < NCHUNK - 1)
            def _():
                for cp in t_copies(cix + 1, nbank):
                    cp.start()
                u_copy(cix + 1, nbank).start()

            for cp in t_copies(cix, bank):
                cp.wait()
            u_copy(cix, bank).wait()

            def pair_body(p, carry2):
                gp = cix * C + p
                tb = cix * CT + p * T
                uraw = plsc.load_gather(uraw_v, [jnp.full((16,), gp, jnp.int32)])
                uhb = ((uraw >> 12) & 1) * EMB_DIM
                pv = jnp.full((16,), p, jnp.int32)
                t1raw = plsc.load_gather(
                    traw_v, [jnp.full((16,), tb, jnp.int32) + lane])
                t2raw = plsc.load_gather(
                    traw_v, [jnp.full((16,), tb + LANES, jnp.int32) + lane],
                    mask=m5)
                q1 = jnp.full((16,), p * T, jnp.int32) + lane
                q2 = jnp.full((16,), p * T + LANES, jnp.int32) + lane
                q2m = jnp.where(m5, q2, 0)
                tb1 = ((t1raw >> 12) & 1) * EMB_DIM
                tb2 = ((t2raw >> 12) & 1) * EMB_DIM

                def dot_body(d, accs):
                    a1, a2 = accs
                    ub = plsc.load_gather(ubuf_v.at[bank], [pv, uhb + d])
                    c1 = plsc.load_gather(tbuf_v.at[bank], [q1, tb1 + d])
                    c2 = plsc.load_gather(tbuf_v.at[bank], [q2m, tb2 + d])
                    return (a1 + ub * c1, a2 + ub * c2)

                z = jnp.zeros((16,), jnp.float32)
                a1, a2 = lax.fori_loop(0, EMB_DIM, dot_body, (z, z), unroll=8)
                plsc.store_scatter(sc_v, [q1], a1)
                plsc.store_scatter(sc_v, [q2m], a2, mask=m5)
                return carry2

            lax.fori_loop(0, C, pair_body, 0, unroll=False)
            pltpu.sync_copy(
                sc_v, out_hbm.at[pl.ds((base + cix * C) * T, CT)])
            return carry

        lax.fori_loop(0, NCHUNK, chunk_body, 0, unroll=False)

    return k(pu_raw, pu_pack, t_raw, t_pack, uc, vc)


def _tc_loss(scores):
    B = scores.shape[0]
    blk = 2048
    grid = B // blk

    def body(s_ref, pos_ref, neg_ref):
        g = pl.program_id(0)
        x = s_ref[...]
        xc = jnp.clip(x, -6.0, 6.0)
        col = lax.broadcasted_iota(jnp.int32, x.shape, 1)
        ispos = col == 0
        isneg = (col >= 1) & (col < T)
        elem = jnp.log1p(jnp.exp(jnp.where(ispos, -xc, xc)))
        pos_p = jnp.sum(jnp.where(ispos, elem, 0.0))
        neg_p = jnp.sum(jnp.where(isneg, elem, 0.0))

        @pl.when(g == 0)
        def _():
            pos_ref[...] = jnp.zeros((1, 1), jnp.float32)
            neg_ref[...] = jnp.zeros((1, 1), jnp.float32)

        pos_ref[...] += jnp.full((1, 1), pos_p, jnp.float32)
        neg_ref[...] += jnp.full((1, 1), neg_p, jnp.float32)

    pos, neg = pl.pallas_call(
        body,
        grid=(grid,),
        in_specs=[pl.BlockSpec((blk, T), lambda g: (g, 0))],
        out_specs=[pl.BlockSpec((1, 1), lambda g: (0, 0)),
                   pl.BlockSpec((1, 1), lambda g: (0, 0))],
        out_shape=[jax.ShapeDtypeStruct((1, 1), jnp.float32)] * 2,
    )(scores)
    return pos[0, 0], neg[0, 0]


@jax.jit
def kernel(pos_u, pos_v, neg_v, u_weight, v_weight):
    B = pos_u.shape[0]
    tgt = jnp.concatenate([pos_v[:, None], neg_v], axis=1).reshape(B * T)
    uc = _compact(u_weight.T)
    vc = _compact(v_weight.T)
    scores = _sc_scores(pos_u, _pack_index(pos_u), tgt, _pack_index(tgt),
                        uc, vc, B)
    return _tc_loss(scores.reshape(B, T))


# speedup vs baseline: 1.3692x; 1.3692x over previous
import functools

import jax
import jax.numpy as jnp
from jax import lax
from jax.experimental import pallas as pl
from jax.experimental.pallas import tpu as pltpu
from jax.experimental.pallas import tpu_sc as plsc

EMB_DIM = 64
NEG = 20
T = NEG + 1
LANES = 16
NC, NS = 2, 16
NW = NC * NS
C = 16
GB = 128
VBLK = 8192


def _pack_index(r):
    return (r >> 13) * (VBLK // 2) + (r & (VBLK // 2 - 1))


def _compact(wt):
    rows = wt.shape[1]
    grid = pl.cdiv(rows, VBLK)

    def body(in_ref, out_ref):
        x = in_ref[...]
        out_ref[:, 0:EMB_DIM] = x[:, 0:VBLK // 2].T
        out_ref[:, EMB_DIM:2 * EMB_DIM] = x[:, VBLK // 2:VBLK].T

    return pl.pallas_call(
        body,
        grid=(grid,),
        in_specs=[pl.BlockSpec((EMB_DIM, VBLK), lambda g: (0, g))],
        out_specs=pl.BlockSpec((VBLK // 2, 2 * EMB_DIM), lambda g: (g, 0)),
        out_shape=jax.ShapeDtypeStruct((grid * (VBLK // 2), 2 * EMB_DIM),
                                       jnp.float32),
    )(wt)


def _sc_scores(pu_raw, pu_pack, t_raw, t_pack, uc, vc, B):
    PW = B // NW
    NCHUNK = PW // C
    CT = C * T

    mesh = plsc.VectorSubcoreMesh(core_axis_name="c", subcore_axis_name="s")

    @functools.partial(
        pl.kernel,
        out_type=jax.ShapeDtypeStruct((B * T,), jnp.float32),
        mesh=mesh,
        scratch_types=[
            pltpu.VMEM((PW,), jnp.int32),
            pltpu.VMEM((PW,), jnp.int32),
            pltpu.VMEM((PW * T,), jnp.int32),
            pltpu.VMEM((PW * T,), jnp.int32),
            pltpu.VMEM((2, C, 2 * EMB_DIM), jnp.float32),
            pltpu.VMEM((2, CT, 2 * EMB_DIM), jnp.float32),
            pltpu.VMEM((CT,), jnp.float32),
            pltpu.SemaphoreType.DMA,
            pltpu.SemaphoreType.DMA,
        ],
        compiler_params=pltpu.CompilerParams(
            needs_layout_passes=False, use_tc_tiling_on_sc=True),
    )
    def k(puraw_hbm, pupack_hbm, traw_hbm, tpack_hbm, uc_hbm, vc_hbm, out_hbm,
          uraw_v, upack_v, traw_v, tpack_v, ubuf_v, tbuf_v, sc_v, sem_u, sem_t):
        wid = lax.axis_index("s") * NC + lax.axis_index("c")
        base = wid * PW
        pltpu.sync_copy(puraw_hbm.at[pl.ds(base, PW)], uraw_v)
        pltpu.sync_copy(pupack_hbm.at[pl.ds(base, PW)], upack_v)
        pltpu.sync_copy(traw_hbm.at[pl.ds(base * T, PW * T)], traw_v)
        pltpu.sync_copy(tpack_hbm.at[pl.ds(base * T, PW * T)], tpack_v)

        lane = lax.iota(jnp.int32, 16)
        last = lane == (LANES - 1)
        nfull, rem = CT // GB, CT % GB

        def t_copies(cix, bank):
            cs = []
            for b in range(nfull):
                cs.append(pltpu.make_async_copy(
                    vc_hbm.at[tpack_v.at[pl.ds(cix * CT + b * GB, GB)]],
                    tbuf_v.at[bank, pl.ds(b * GB, GB)], sem_t))
            if rem:
                cs.append(pltpu.make_async_copy(
                    vc_hbm.at[tpack_v.at[pl.ds(cix * CT + nfull * GB, rem)]],
                    tbuf_v.at[bank, pl.ds(nfull * GB, rem)], sem_t))
            return cs

        def u_copy(cix, bank):
            return pltpu.make_async_copy(
                uc_hbm.at[upack_v.at[pl.ds(cix * C, C)]],
                ubuf_v.at[bank], sem_u)

        for cp in t_copies(0, 0):
            cp.start()
        u_copy(0, 0).start()

        def chunk_body(cix, carry):
            bank = lax.rem(cix, 2)
            nbank = 1 - bank

            @pl.when(cix < NCHUNK - 1)
            def _():
                for cp in t_copies(cix + 1, nbank):
                    cp.start()
                u_copy(cix + 1, nbank).start()

            for cp in t_copies(cix, bank):
                cp.wait()
            u_copy(cix, bank).wait()

            def pair_body(p, carry2):
                gp = cix * C + p
                uraw = plsc.load_gather(uraw_v, [jnp.full((16,), gp, jnp.int32)])
                ucol = ((uraw >> 12) & 1) * EMB_DIM + lane
                pv = jnp.full((16,), p, jnp.int32)
                u0 = plsc.load_gather(ubuf_v.at[bank], [pv, ucol])
                u1 = plsc.load_gather(ubuf_v.at[bank], [pv, ucol + 16])
                u2 = plsc.load_gather(ubuf_v.at[bank], [pv, ucol + 32])
                u3 = plsc.load_gather(ubuf_v.at[bank], [pv, ucol + 48])
                for t in range(T):
                    q = p * T + t
                    traw = plsc.load_gather(
                        traw_v, [jnp.full((16,), cix * CT + q, jnp.int32)])
                    tcol = ((traw >> 12) & 1) * EMB_DIM + lane
                    qv = jnp.full((16,), q, jnp.int32)
                    t0 = plsc.load_gather(tbuf_v.at[bank], [qv, tcol])
                    t1 = plsc.load_gather(tbuf_v.at[bank], [qv, tcol + 16])
                    t2 = plsc.load_gather(tbuf_v.at[bank], [qv, tcol + 32])
                    t3 = plsc.load_gather(tbuf_v.at[bank], [qv, tcol + 48])
                    part = u0 * t0 + u1 * t1 + u2 * t2 + u3 * t3
                    cum = plsc.cumsum(part)
                    plsc.store_scatter(sc_v, [qv], cum, mask=last)
                return carry2

            lax.fori_loop(0, C, pair_body, 0, unroll=False)
            pltpu.sync_copy(
                sc_v, out_hbm.at[pl.ds((base + cix * C) * T, CT)])
            return carry

        lax.fori_loop(0, NCHUNK, chunk_body, 0, unroll=False)

    return k(pu_raw, pu_pack, t_raw, t_pack, uc, vc)


def _tc_loss(scores):
    B = scores.shape[0]
    blk = 2048
    grid = B // blk

    def body(s_ref, pos_ref, neg_ref):
        g = pl.program_id(0)
        x = s_ref[...]
        xc = jnp.clip(x, -6.0, 6.0)
        col = lax.broadcasted_iota(jnp.int32, x.shape, 1)
        ispos = col == 0
        isneg = (col >= 1) & (col < T)
        elem = jnp.log1p(jnp.exp(jnp.where(ispos, -xc, xc)))
        pos_p = jnp.sum(jnp.where(ispos, elem, 0.0))
        neg_p = jnp.sum(jnp.where(isneg, elem, 0.0))

        @pl.when(g == 0)
        def _():
            pos_ref[...] = jnp.zeros((1, 1), jnp.float32)
            neg_ref[...] = jnp.zeros((1, 1), jnp.float32)

        pos_ref[...] += jnp.full((1, 1), pos_p, jnp.float32)
        neg_ref[...] += jnp.full((1, 1), neg_p, jnp.float32)

    pos, neg = pl.pallas_call(
        body,
        grid=(grid,),
        in_specs=[pl.BlockSpec((blk, T), lambda g: (g, 0))],
        out_specs=[pl.BlockSpec((1, 1), lambda g: (0, 0)),
                   pl.BlockSpec((1, 1), lambda g: (0, 0))],
        out_shape=[jax.ShapeDtypeStruct((1, 1), jnp.float32)] * 2,
    )(scores)
    return pos[0, 0], neg[0, 0]


@jax.jit
def kernel(pos_u, pos_v, neg_v, u_weight, v_weight):
    B = pos_u.shape[0]
    tgt = jnp.concatenate([pos_v[:, None], neg_v], axis=1).reshape(B * T)
    uc = _compact(u_weight.T)
    vc = _compact(v_weight.T)
    scores = _sc_scores(pos_u, _pack_index(pos_u), tgt, _pack_index(tgt),
                        uc, vc, B)
    return _tc_loss(scores.reshape(B, T))


# speedup vs baseline: 1.5994x; 1.1682x over previous
import functools

import jax
import jax.numpy as jnp
from jax import lax
from jax.experimental import pallas as pl
from jax.experimental.pallas import tpu as pltpu
from jax.experimental.pallas import tpu_sc as plsc

EMB_DIM = 64
NEG = 20
T = NEG + 1
LANES = 16
NC, NS = 2, 16
NW = NC * NS
C = 16
GB = 128
VBLK = 8192


def _lane_permute(x, idx):
    return lax.gather(
        x, idx[:, None],
        dimension_numbers=lax.GatherDimensionNumbers(
            offset_dims=(), collapsed_slice_dims=(0,), start_index_map=(0,)),
        slice_sizes=(1,),
        mode=lax.GatherScatterMode.PROMISE_IN_BOUNDS)


def _pack_index(r):
    return (r >> 13) * (VBLK // 2) + (r & (VBLK // 2 - 1))


def _compact2(ut, vt):
    rows = ut.shape[1]
    grid1 = pl.cdiv(rows, VBLK)

    def body(u_ref, v_ref, uo_ref, vo_ref):
        g = pl.program_id(0)

        def write(src, dst):
            dst[:, 0:EMB_DIM] = src[:, 0:VBLK // 2].T
            dst[:, EMB_DIM:2 * EMB_DIM] = src[:, VBLK // 2:VBLK].T

        @pl.when(g < grid1)
        def _():
            write(u_ref[...], uo_ref)

        @pl.when(g >= grid1)
        def _():
            write(v_ref[...], vo_ref)

    oshape = jax.ShapeDtypeStruct((grid1 * (VBLK // 2), 2 * EMB_DIM),
                                  jnp.float32)
    return pl.pallas_call(
        body,
        grid=(2 * grid1,),
        in_specs=[
            pl.BlockSpec((EMB_DIM, VBLK),
                         lambda g: (0, jnp.minimum(g, grid1 - 1))),
            pl.BlockSpec((EMB_DIM, VBLK),
                         lambda g: (0, jnp.maximum(g - grid1, 0))),
        ],
        out_specs=[
            pl.BlockSpec((VBLK // 2, 2 * EMB_DIM),
                         lambda g: (jnp.minimum(g, grid1 - 1), 0)),
            pl.BlockSpec((VBLK // 2, 2 * EMB_DIM),
                         lambda g: (jnp.maximum(g - grid1, 0), 0)),
        ],
        out_shape=[oshape, oshape],
    )(ut, vt)


def _sc_scores(pu_raw, pu_pack, t_raw, t_pack, uc, vc, B):
    PW = B // NW
    NCHUNK = PW // C
    CT = C * T

    mesh = plsc.VectorSubcoreMesh(core_axis_name="c", subcore_axis_name="s")

    @functools.partial(
        pl.kernel,
        out_type=jax.ShapeDtypeStruct((B * T,), jnp.float32),
        mesh=mesh,
        scratch_types=[
            pltpu.VMEM((PW,), jnp.int32),
            pltpu.VMEM((PW,), jnp.int32),
            pltpu.VMEM((PW * T,), jnp.int32),
            pltpu.VMEM((PW * T,), jnp.int32),
            pltpu.VMEM((2, C, 2 * EMB_DIM), jnp.float32),
            pltpu.VMEM((2, CT, 2 * EMB_DIM), jnp.float32),
            pltpu.VMEM((CT,), jnp.float32),
            pltpu.SemaphoreType.DMA,
            pltpu.SemaphoreType.DMA,
        ],
        compiler_params=pltpu.CompilerParams(
            needs_layout_passes=False, use_tc_tiling_on_sc=True),
    )
    def k(puraw_hbm, pupack_hbm, traw_hbm, tpack_hbm, uc_hbm, vc_hbm, out_hbm,
          uraw_v, upack_v, traw_v, tpack_v, ubuf_v, tbuf_v, sc_v, sem_u, sem_t):
        wid = lax.axis_index("s") * NC + lax.axis_index("c")
        base = wid * PW
        pltpu.sync_copy(puraw_hbm.at[pl.ds(base, PW)], uraw_v)
        pltpu.sync_copy(pupack_hbm.at[pl.ds(base, PW)], upack_v)
        pltpu.sync_copy(traw_hbm.at[pl.ds(base * T, PW * T)], traw_v)
        pltpu.sync_copy(tpack_hbm.at[pl.ds(base * T, PW * T)], tpack_v)

        lane = lax.iota(jnp.int32, 16)
        last = lane == (LANES - 1)
        nfull, rem = CT // GB, CT % GB

        def t_copies(cix, bank):
            cs = []
            for b in range(nfull):
                cs.append(pltpu.make_async_copy(
                    vc_hbm.at[tpack_v.at[pl.ds(cix * CT + b * GB, GB)]],
                    tbuf_v.at[bank, pl.ds(b * GB, GB)], sem_t))
            if rem:
                cs.append(pltpu.make_async_copy(
                    vc_hbm.at[tpack_v.at[pl.ds(cix * CT + nfull * GB, rem)]],
                    tbuf_v.at[bank, pl.ds(nfull * GB, rem)], sem_t))
            return cs

        def u_copy(cix, bank):
            return pltpu.make_async_copy(
                uc_hbm.at[upack_v.at[pl.ds(cix * C, C)]],
                ubuf_v.at[bank], sem_u)

        for cp in t_copies(0, 0):
            cp.start()
        u_copy(0, 0).start()

        def chunk_body(cix, carry):
            bank = lax.rem(cix, 2)
            nbank = 1 - bank

            @pl.when(cix < NCHUNK - 1)
            def _():
                for cp in t_copies(cix + 1, nbank):
                    cp.start()
                u_copy(cix + 1, nbank).start()

            for cp in t_copies(cix, bank):
                cp.wait()
            u_copy(cix, bank).wait()

            def pair_body(p, carry2):
                gp = cix * C + p
                uraw = plsc.load_gather(uraw_v, [jnp.full((16,), gp, jnp.int32)])
                ucol = ((uraw >> 12) & 1) * EMB_DIM + lane
                pv = jnp.full((16,), p, jnp.int32)
                u0 = plsc.load_gather(ubuf_v.at[bank], [pv, ucol])
                u1 = plsc.load_gather(ubuf_v.at[bank], [pv, ucol + 16])
                u2 = plsc.load_gather(ubuf_v.at[bank], [pv, ucol + 32])
                u3 = plsc.load_gather(ubuf_v.at[bank], [pv, ucol + 48])
                def t_loads(t):
                    q = p * T + t
                    traw = plsc.load_gather(
                        traw_v, [jnp.full((16,), cix * CT + q, jnp.int32)])
                    tcol = ((traw >> 12) & 1) * EMB_DIM + lane
                    qv = jnp.full((16,), q, jnp.int32)
                    return (qv,
                            plsc.load_gather(tbuf_v.at[bank], [qv, tcol]),
                            plsc.load_gather(tbuf_v.at[bank], [qv, tcol + 16]),
                            plsc.load_gather(tbuf_v.at[bank], [qv, tcol + 32]),
                            plsc.load_gather(tbuf_v.at[bank], [qv, tcol + 48]))

                cur = t_loads(0)
                for t in range(T):
                    nxt = t_loads(t + 1) if t + 1 < T else cur
                    qv, t0, t1, t2, t3 = cur
                    part = u0 * t0 + u1 * t1 + u2 * t2 + u3 * t3
                    cum = plsc.cumsum(part)
                    plsc.store_scatter(sc_v, [qv], cum, mask=last)
                    cur = nxt
                return carry2

            lax.fori_loop(0, C, pair_body, 0, unroll=False)
            pltpu.sync_copy(
                sc_v, out_hbm.at[pl.ds((base + cix * C) * T, CT)])
            return carry

        lax.fori_loop(0, NCHUNK, chunk_body, 0, unroll=False)

    return k(pu_raw, pu_pack, t_raw, t_pack, uc, vc)


def _tc_loss(scores):
    n = scores.shape[0]
    grid = 8
    blk = n // grid

    def body(s_ref, pos_ref, neg_ref):
        g = pl.program_id(0)
        x = s_ref[...]
        xc = jnp.clip(x, -6.0, 6.0)
        idx = g * blk + lax.broadcasted_iota(jnp.int32, x.shape, 0)
        ispos = lax.rem(idx, T) == 0
        elem = jnp.log1p(jnp.exp(jnp.where(ispos, -xc, xc)))
        pos_p = jnp.sum(jnp.where(ispos, elem, 0.0))
        neg_p = jnp.sum(jnp.where(ispos, 0.0, elem))

        @pl.when(g == 0)
        def _():
            pos_ref[...] = jnp.zeros((1, 1), jnp.float32)
            neg_ref[...] = jnp.zeros((1, 1), jnp.float32)

        pos_ref[...] += jnp.full((1, 1), pos_p, jnp.float32)
        neg_ref[...] += jnp.full((1, 1), neg_p, jnp.float32)

    pos, neg = pl.pallas_call(
        body,
        grid=(grid,),
        in_specs=[pl.BlockSpec((blk,), lambda g: (g,))],
        out_specs=[pl.BlockSpec((1, 1), lambda g: (0, 0)),
                   pl.BlockSpec((1, 1), lambda g: (0, 0))],
        out_shape=[jax.ShapeDtypeStruct((1, 1), jnp.float32)] * 2,
    )(scores)
    return pos[0, 0], neg[0, 0]


@jax.jit
def kernel(pos_u, pos_v, neg_v, u_weight, v_weight):
    B = pos_u.shape[0]
    tgt = jnp.concatenate([pos_v[:, None], neg_v], axis=1).reshape(B * T)
    uc, vc = _compact2(u_weight.T, v_weight.T)
    scores = _sc_scores(pos_u, _pack_index(pos_u), tgt, _pack_index(tgt),
                        uc, vc, B)
    return _tc_loss(scores)
